# Initial kernel scaffold; baseline (speedup 1.0000x reference)
import jax
import jax.numpy as jnp
from jax.experimental import pallas as pl


def kernel(words, postags, word_table, pos_table, gamma, beta):
    raise NotImplementedError("write your pallas kernel here")


# speedup vs baseline: 1.5677x; 1.5677x over previous
import functools

import jax
import jax.numpy as jnp
from jax import lax
from jax.experimental import pallas as pl
from jax.experimental.pallas import tpu as pltpu
from jax.experimental.pallas import tpu_sc as plsc

DW, DP, D = 128, 64, 192
VOCAB_POS = 64
EPS = 1e-6
NC, NS = 2, 16
NW = NC * NS
B, L = 4096, 50
N = B * L
TPW = N // NW
CB = 128
NCHUNK = TPW // CB
NJW = DW // 16
NJP = DP // 16
NJ = D // 16


def _sc_body(words_hbm, postags_hbm, wtab_hbm, ptab_hbm, gamma_hbm, beta_hbm,
             out_hbm, widx_v, pidx_v, wrows_v, prows_v, obuf_v, gamma_v,
             beta_v, wsem, psem):
    wid = lax.axis_index("s") * NC + lax.axis_index("c")
    base = wid * TPW

    pltpu.sync_copy(gamma_hbm, gamma_v)
    pltpu.sync_copy(beta_hbm, beta_v)

    @pl.loop(0, NCHUNK)
    def _chunk(c):
        cbase = base + c * CB
        pltpu.sync_copy(words_hbm.at[pl.ds(cbase, CB)], widx_v)
        pltpu.sync_copy(postags_hbm.at[pl.ds(cbase, CB)], pidx_v)
        wcp = pltpu.async_copy(wtab_hbm.at[widx_v], wrows_v, wsem)
        pcp = pltpu.async_copy(ptab_hbm.at[pidx_v], prows_v, psem)
        wcp.wait()
        pcp.wait()

        @pl.loop(0, CB)
        def _tok(t):
            xs = [wrows_v[t, pl.ds(16 * j, 16)] for j in range(NJW)]
            ps = [prows_v[t, pl.ds(16 * j, 16)] for j in range(NJP)]
            vals = xs + ps
            s = vals[0]
            sq = vals[0] * vals[0]
            for v in vals[1:]:
                s = s + v
                sq = sq + v * v
            mean = jnp.sum(s) * (1.0 / D)
            var = jnp.sum(sq) * (1.0 / D) - mean * mean
            meanv = lax.broadcast(mean, (16,))
            xv = lax.broadcast(var + EPS, (16,))
            i = plsc.bitcast(xv, jnp.int32)
            i = 0x5F3759DF - lax.shift_right_arithmetic(i, 1)
            y = plsc.bitcast(i, jnp.float32)
            y = y * (1.5 - 0.5 * xv * y * y)
            y = y * (1.5 - 0.5 * xv * y * y)
            for j in range(NJ):
                g = gamma_v[pl.ds(16 * j, 16)]
                bt = beta_v[pl.ds(16 * j, 16)]
                obuf_v[t, pl.ds(16 * j, 16)] = (vals[j] - meanv) * y * g + bt

        pltpu.sync_copy(obuf_v, out_hbm.at[pl.ds(cbase, CB)])


@jax.jit
def kernel(words, postags, word_table, pos_table, gamma, beta):
    words_f = words.reshape(-1).astype(jnp.int32)
    postags_f = postags.reshape(-1).astype(jnp.int32)
    ptab_pad = jnp.pad(pos_table, ((0, 0), (0, DW - DP)))
    mesh = plsc.VectorSubcoreMesh(core_axis_name="c", subcore_axis_name="s",
                                  num_cores=NC, num_subcores=NS)
    run = pl.kernel(
        _sc_body,
        out_type=jax.ShapeDtypeStruct((N, D), jnp.float32),
        mesh=mesh,
        compiler_params=pltpu.CompilerParams(needs_layout_passes=False),
        scratch_types=[
            pltpu.VMEM((CB,), jnp.int32),
            pltpu.VMEM((CB,), jnp.int32),
            pltpu.VMEM((CB, DW), jnp.float32),
            pltpu.VMEM((CB, DW), jnp.float32),
            pltpu.VMEM((CB, D), jnp.float32),
            pltpu.VMEM((D,), jnp.float32),
            pltpu.VMEM((D,), jnp.float32),
            pltpu.SemaphoreType.DMA,
            pltpu.SemaphoreType.DMA,
        ],
    )
    out = run(words_f, postags_f, word_table, ptab_pad, gamma, beta)
    return out.reshape(B, L, D)


# speedup vs baseline: 1.5694x; 1.0011x over previous
import functools

import jax
import jax.numpy as jnp
from jax import lax
from jax.experimental import pallas as pl
from jax.experimental.pallas import tpu as pltpu
from jax.experimental.pallas import tpu_sc as plsc

DW, DP, D = 128, 64, 192
VOCAB_POS = 64
EPS = 1e-6
NC, NS = 2, 16
NW = NC * NS
B, L = 4096, 50
N = B * L
TPW = N // NW
CB = 128
NCHUNK = TPW // CB
NJW = DW // 16
NJP = DP // 16
NJ = D // 16


def _sc_body(words_hbm, postags_hbm, wtab_hbm, ptab_hbm, gamma_hbm, beta_hbm,
             out_hbm, widx_v, pidx_v, wrows_v, prows_v, obuf_v, gamma_v,
             beta_v, wsem, psem):
    wid = lax.axis_index("s") * NC + lax.axis_index("c")
    base = wid * TPW

    pltpu.sync_copy(gamma_hbm, gamma_v)
    pltpu.sync_copy(beta_hbm, beta_v)

    @pl.loop(0, NCHUNK)
    def _chunk(c):
        cbase = base + c * CB
        pltpu.sync_copy(words_hbm.at[pl.ds(cbase, CB)], widx_v)
        pltpu.sync_copy(postags_hbm.at[pl.ds(cbase, CB)], pidx_v)
        wcp = pltpu.async_copy(wtab_hbm.at[widx_v], wrows_v, wsem)
        pcp = pltpu.async_copy(ptab_hbm.at[pidx_v], prows_v, psem)
        wcp.wait()
        pcp.wait()

        @pl.loop(0, CB, unroll=8)
        def _tok(t):
            xs = [wrows_v[t, pl.ds(16 * j, 16)] for j in range(NJW)]
            ps = [prows_v[t, pl.ds(16 * j, 16)] for j in range(NJP)]
            vals = xs + ps
            s = vals[0]
            sq = vals[0] * vals[0]
            for v in vals[1:]:
                s = s + v
                sq = sq + v * v
            mean = jnp.sum(s) * (1.0 / D)
            var = jnp.sum(sq) * (1.0 / D) - mean * mean
            meanv = lax.broadcast(mean, (16,))
            xv = lax.broadcast(var + EPS, (16,))
            i = plsc.bitcast(xv, jnp.int32)
            i = 0x5F3759DF - lax.shift_right_arithmetic(i, 1)
            y = plsc.bitcast(i, jnp.float32)
            y = y * (1.5 - 0.5 * xv * y * y)
            y = y * (1.5 - 0.5 * xv * y * y)
            for j in range(NJ):
                g = gamma_v[pl.ds(16 * j, 16)]
                bt = beta_v[pl.ds(16 * j, 16)]
                obuf_v[t, pl.ds(16 * j, 16)] = (vals[j] - meanv) * y * g + bt

        pltpu.sync_copy(obuf_v, out_hbm.at[pl.ds(cbase, CB)])


@jax.jit
def kernel(words, postags, word_table, pos_table, gamma, beta):
    words_f = words.reshape(-1).astype(jnp.int32)
    postags_f = postags.reshape(-1).astype(jnp.int32)
    ptab_pad = jnp.pad(pos_table, ((0, 0), (0, DW - DP)))
    mesh = plsc.VectorSubcoreMesh(core_axis_name="c", subcore_axis_name="s",
                                  num_cores=NC, num_subcores=NS)
    run = pl.kernel(
        _sc_body,
        out_type=jax.ShapeDtypeStruct((N, D), jnp.float32),
        mesh=mesh,
        compiler_params=pltpu.CompilerParams(needs_layout_passes=False),
        scratch_types=[
            pltpu.VMEM((CB,), jnp.int32),
            pltpu.VMEM((CB,), jnp.int32),
            pltpu.VMEM((CB, DW), jnp.float32),
            pltpu.VMEM((CB, DW), jnp.float32),
            pltpu.VMEM((CB, D), jnp.float32),
            pltpu.VMEM((D,), jnp.float32),
            pltpu.VMEM((D,), jnp.float32),
            pltpu.SemaphoreType.DMA,
            pltpu.SemaphoreType.DMA,
        ],
    )
    out = run(words_f, postags_f, word_table, ptab_pad, gamma, beta)
    return out.reshape(B, L, D)


# speedup vs baseline: 2.5024x; 1.5945x over previous
import functools

import jax
import jax.numpy as jnp
from jax import lax
from jax.experimental import pallas as pl
from jax.experimental.pallas import tpu as pltpu
from jax.experimental.pallas import tpu_sc as plsc

DW, DP, D = 128, 64, 192
VOCAB_POS = 64
EPS = 1e-6
NC, NS = 2, 16
NW = NC * NS
B, L = 4096, 50
N = B * L
TPW = N // NW
CB = 128
NCHUNK = TPW // CB
NJW = DW // 16
NJP = DP // 16
NJ = D // 16


def _sc_body(words_hbm, postags_hbm, wtab_hbm, ptab_hbm, gamma_hbm, beta_hbm,
             out_hbm, widx_v, pidx_v, wrows_v, prows_v, obuf_v, gamma_v,
             beta_v, wsem, psem):
    wid = lax.axis_index("s") * NC + lax.axis_index("c")
    base = wid * TPW

    pltpu.sync_copy(gamma_hbm, gamma_v)
    pltpu.sync_copy(beta_hbm, beta_v)

    @pl.loop(0, NCHUNK)
    def _chunk(c):
        cbase = base + c * CB
        pltpu.sync_copy(words_hbm.at[pl.ds(cbase, CB)], widx_v)
        pltpu.sync_copy(postags_hbm.at[pl.ds(cbase, CB)], pidx_v)
        wcp = pltpu.async_copy(wtab_hbm.at[widx_v], wrows_v, wsem)
        pcp = pltpu.async_copy(ptab_hbm.at[pidx_v], prows_v, psem)
        wcp.wait()
        pcp.wait()

        @plsc.parallel_loop(0, CB, unroll=8)
        def _tok(t):
            xs = [wrows_v[t, pl.ds(16 * j, 16)] for j in range(NJW)]
            ps = [prows_v[t, pl.ds(16 * j, 16)] for j in range(NJP)]
            vals = xs + ps
            s = vals[0]
            sq = vals[0] * vals[0]
            for v in vals[1:]:
                s = s + v
                sq = sq + v * v
            mean = jnp.sum(s) * (1.0 / D)
            var = jnp.sum(sq) * (1.0 / D) - mean * mean
            meanv = lax.broadcast(mean, (16,))
            xv = lax.broadcast(var + EPS, (16,))
            i = plsc.bitcast(xv, jnp.int32)
            i = 0x5F3759DF - lax.shift_right_arithmetic(i, 1)
            y = plsc.bitcast(i, jnp.float32)
            y = y * (1.5 - 0.5 * xv * y * y)
            y = y * (1.5 - 0.5 * xv * y * y)
            for j in range(NJ):
                g = gamma_v[pl.ds(16 * j, 16)]
                bt = beta_v[pl.ds(16 * j, 16)]
                obuf_v[t, pl.ds(16 * j, 16)] = (vals[j] - meanv) * y * g + bt

        pltpu.sync_copy(obuf_v, out_hbm.at[pl.ds(cbase, CB)])


@jax.jit
def kernel(words, postags, word_table, pos_table, gamma, beta):
    words_f = words.reshape(-1).astype(jnp.int32)
    postags_f = postags.reshape(-1).astype(jnp.int32)
    ptab_pad = jnp.pad(pos_table, ((0, 0), (0, DW - DP)))
    mesh = plsc.VectorSubcoreMesh(core_axis_name="c", subcore_axis_name="s",
                                  num_cores=NC, num_subcores=NS)
    run = pl.kernel(
        _sc_body,
        out_type=jax.ShapeDtypeStruct((N, D), jnp.float32),
        mesh=mesh,
        compiler_params=pltpu.CompilerParams(needs_layout_passes=False),
        scratch_types=[
            pltpu.VMEM((CB,), jnp.int32),
            pltpu.VMEM((CB,), jnp.int32),
            pltpu.VMEM((CB, DW), jnp.float32),
            pltpu.VMEM((CB, DW), jnp.float32),
            pltpu.VMEM((CB, D), jnp.float32),
            pltpu.VMEM((D,), jnp.float32),
            pltpu.VMEM((D,), jnp.float32),
            pltpu.SemaphoreType.DMA,
            pltpu.SemaphoreType.DMA,
        ],
    )
    out = run(words_f, postags_f, word_table, ptab_pad, gamma, beta)
    return out.reshape(B, L, D)


# speedup vs baseline: 2.7182x; 1.0862x over previous
import functools

import jax
import jax.numpy as jnp
from jax import lax
from jax.experimental import pallas as pl
from jax.experimental.pallas import tpu as pltpu
from jax.experimental.pallas import tpu_sc as plsc

DW, DP, D = 128, 64, 192
VOCAB_POS = 64
EPS = 1e-6
NC, NS = 2, 16
NW = NC * NS
B, L = 4096, 50
N = B * L
TPW = N // NW
CB = 128
NCHUNK = TPW // CB
NJW = DW // 16
NJP = DP // 16
NJ = D // 16


def _sc_body(words_hbm, postags_hbm, wtab_hbm, ptab_hbm, gamma_hbm, beta_hbm,
             out_hbm, widx0, widx1, pidx0, pidx1, wrows0, wrows1, prows0,
             prows1, obuf_v, gamma_v, beta_v, sem0, sem1):
    wid = lax.axis_index("s") * NC + lax.axis_index("c")
    base = wid * TPW

    pltpu.sync_copy(gamma_hbm, gamma_v)
    pltpu.sync_copy(beta_hbm, beta_v)

    slot0 = (widx0, pidx0, wrows0, prows0, sem0)
    slot1 = (widx1, pidx1, wrows1, prows1, sem1)

    def _issue(cbase, slot):
        widx, pidx, wrows, prows, sem = slot
        pltpu.sync_copy(words_hbm.at[pl.ds(cbase, CB)], widx)
        pltpu.sync_copy(postags_hbm.at[pl.ds(cbase, CB)], pidx)
        pltpu.async_copy(wtab_hbm.at[widx], wrows, sem)
        pltpu.async_copy(ptab_hbm.at[pidx], prows, sem)

    def _drain(slot):
        widx, pidx, wrows, prows, sem = slot
        pltpu.make_async_copy(wtab_hbm.at[widx], wrows, sem).wait()
        pltpu.make_async_copy(ptab_hbm.at[pidx], prows, sem).wait()

    def _compute(cbase, slot):
        _, _, wrows_v, prows_v, _ = slot

        @plsc.parallel_loop(0, CB, unroll=8)
        def _tok(t):
            xs = [wrows_v[t, pl.ds(16 * j, 16)] for j in range(NJW)]
            ps = [prows_v[t, pl.ds(16 * j, 16)] for j in range(NJP)]
            vals = xs + ps
            s = vals[0]
            sq = vals[0] * vals[0]
            for v in vals[1:]:
                s = s + v
                sq = sq + v * v
            mean = jnp.sum(s) * (1.0 / D)
            var = jnp.sum(sq) * (1.0 / D) - mean * mean
            meanv = lax.broadcast(mean, (16,))
            xv = lax.broadcast(var + EPS, (16,))
            i = plsc.bitcast(xv, jnp.int32)
            i = 0x5F3759DF - lax.shift_right_arithmetic(i, 1)
            y = plsc.bitcast(i, jnp.float32)
            y = y * (1.5 - 0.5 * xv * y * y)
            y = y * (1.5 - 0.5 * xv * y * y)
            for j in range(NJ):
                g = gamma_v[pl.ds(16 * j, 16)]
                bt = beta_v[pl.ds(16 * j, 16)]
                obuf_v[t, pl.ds(16 * j, 16)] = (vals[j] - meanv) * y * g + bt

        pltpu.sync_copy(obuf_v, out_hbm.at[pl.ds(cbase, CB)])

    _issue(base, slot0)

    @pl.loop(0, NCHUNK, step=2)
    def _chunk(c):
        cb0 = base + c * CB
        _issue(cb0 + CB, slot1)
        _drain(slot0)
        _compute(cb0, slot0)

        @pl.when(c + 2 < NCHUNK)
        def _():
            _issue(cb0 + 2 * CB, slot0)

        _drain(slot1)
        _compute(cb0 + CB, slot1)


@jax.jit
def kernel(words, postags, word_table, pos_table, gamma, beta):
    words_f = words.reshape(-1).astype(jnp.int32)
    postags_f = postags.reshape(-1).astype(jnp.int32)
    ptab_pad = jnp.pad(pos_table, ((0, 0), (0, DW - DP)))
    mesh = plsc.VectorSubcoreMesh(core_axis_name="c", subcore_axis_name="s",
                                  num_cores=NC, num_subcores=NS)
    run = pl.kernel(
        _sc_body,
        out_type=jax.ShapeDtypeStruct((N, D), jnp.float32),
        mesh=mesh,
        compiler_params=pltpu.CompilerParams(needs_layout_passes=False),
        scratch_types=[
            pltpu.VMEM((CB,), jnp.int32),
            pltpu.VMEM((CB,), jnp.int32),
            pltpu.VMEM((CB,), jnp.int32),
            pltpu.VMEM((CB,), jnp.int32),
            pltpu.VMEM((CB, DW), jnp.float32),
            pltpu.VMEM((CB, DW), jnp.float32),
            pltpu.VMEM((CB, DW), jnp.float32),
            pltpu.VMEM((CB, DW), jnp.float32),
            pltpu.VMEM((CB, D), jnp.float32),
            pltpu.VMEM((D,), jnp.float32),
            pltpu.VMEM((D,), jnp.float32),
            pltpu.SemaphoreType.DMA,
            pltpu.SemaphoreType.DMA,
        ],
    )
    out = run(words_f, postags_f, word_table, ptab_pad, gamma, beta)
    return out.reshape(B, L, D)


# speedup vs baseline: 2.7208x; 1.0010x over previous
import functools

import jax
import jax.numpy as jnp
from jax import lax
from jax.experimental import pallas as pl
from jax.experimental.pallas import tpu as pltpu
from jax.experimental.pallas import tpu_sc as plsc

DW, DP, D = 128, 64, 192
VOCAB_POS = 64
EPS = 1e-6
NC, NS = 2, 16
NW = NC * NS
B, L = 4096, 50
N = B * L
TPW = N // NW
CB = 128
PB = CB // 2
NCHUNK = TPW // CB
NJW = DW // 16
NJP = DP // 16
NJ = D // 16


def _norm_one(vals, gamma_v, beta_v, ob, row):
    s = vals[0]
    sq = vals[0] * vals[0]
    for v in vals[1:]:
        s = s + v
        sq = sq + v * v
    mean = jnp.sum(s) * (1.0 / D)
    var = jnp.sum(sq) * (1.0 / D) - mean * mean
    meanv = lax.broadcast(mean, (16,))
    xv = lax.broadcast(var + EPS, (16,))
    i = plsc.bitcast(xv, jnp.int32)
    i = 0x5F3759DF - lax.shift_right_arithmetic(i, 1)
    y = plsc.bitcast(i, jnp.float32)
    y = y * (1.5 - 0.5 * xv * y * y)
    y = y * (1.5 - 0.5 * xv * y * y)
    for j in range(NJ):
        g = gamma_v[pl.ds(16 * j, 16)]
        bt = beta_v[pl.ds(16 * j, 16)]
        ob[row, pl.ds(16 * j, 16)] = (vals[j] - meanv) * y * g + bt


def _sc_body(words_hbm, pidx2_hbm, wtab_hbm, ptab2_hbm, gamma_hbm, beta_hbm,
             out_hbm, widx_all, pidx_all, wrows0, wrows1, prows0, prows1,
             obuf0, obuf1, gamma_v, beta_v, sem0, sem1, osem0, osem1):
    wid = lax.axis_index("s") * NC + lax.axis_index("c")
    base = wid * TPW

    pltpu.sync_copy(gamma_hbm, gamma_v)
    pltpu.sync_copy(beta_hbm, beta_v)
    pltpu.sync_copy(words_hbm.at[pl.ds(base, TPW)], widx_all)
    pltpu.sync_copy(pidx2_hbm.at[pl.ds(wid * (TPW // 2), TPW // 2)], pidx_all)

    slot0 = (wrows0, prows0, sem0)
    slot1 = (wrows1, prows1, sem1)

    def _issue(c, slot):
        wrows, prows, sem = slot
        pltpu.async_copy(
            wtab_hbm.at[widx_all.at[pl.ds(c * CB, CB)]], wrows, sem)
        pltpu.async_copy(
            ptab2_hbm.at[pidx_all.at[pl.ds(c * PB, PB)]], prows, sem)

    def _drain(c, slot):
        wrows, prows, sem = slot
        pltpu.make_async_copy(
            wtab_hbm.at[widx_all.at[pl.ds(c * CB, CB)]], wrows, sem).wait()
        pltpu.make_async_copy(
            ptab2_hbm.at[pidx_all.at[pl.ds(c * PB, PB)]], prows, sem).wait()

    def _compute(c, slot, ob, osem, pending):
        wrows_v, prows_v, _ = slot
        if pending is True:
            pltpu.make_async_copy(ob, out_hbm.at[pl.ds(base, CB)],
                                  osem).wait()
        else:
            @pl.when(pending)
            def _():
                pltpu.make_async_copy(ob, out_hbm.at[pl.ds(base, CB)],
                                      osem).wait()

        @plsc.parallel_loop(0, PB, unroll=4)
        def _pair(q):
            x0 = [wrows_v[2 * q, pl.ds(16 * j, 16)] for j in range(NJW)]
            x1 = [wrows_v[2 * q + 1, pl.ds(16 * j, 16)] for j in range(NJW)]
            p01 = [prows_v[q, pl.ds(16 * j, 16)] for j in range(2 * NJP)]
            _norm_one(x0 + p01[:NJP], gamma_v, beta_v, ob, 2 * q)
            _norm_one(x1 + p01[NJP:], gamma_v, beta_v, ob, 2 * q + 1)

        pltpu.async_copy(ob, out_hbm.at[pl.ds(base + c * CB, CB)], osem)

    _issue(0, slot0)

    @pl.loop(0, NCHUNK, step=2)
    def _chunk(c):
        _issue(c + 1, slot1)
        _drain(c, slot0)
        _compute(c, slot0, obuf0, osem0, c > 0)

        @pl.when(c + 2 < NCHUNK)
        def _():
            _issue(c + 2, slot0)

        _drain(c + 1, slot1)
        _compute(c + 1, slot1, obuf1, osem1, c > 0)

    pltpu.make_async_copy(obuf0, out_hbm.at[pl.ds(base, CB)], osem0).wait()
    pltpu.make_async_copy(obuf1, out_hbm.at[pl.ds(base, CB)], osem1).wait()


@jax.jit
def kernel(words, postags, word_table, pos_table, gamma, beta):
    words_f = words.reshape(-1).astype(jnp.int32)
    pp = postags.reshape(-1, 2).astype(jnp.int32)
    pidx2 = pp[:, 0] * VOCAB_POS + pp[:, 1]
    ptab2 = jnp.concatenate(
        [jnp.repeat(pos_table, VOCAB_POS, axis=0),
         jnp.tile(pos_table, (VOCAB_POS, 1))], axis=1)
    mesh = plsc.VectorSubcoreMesh(core_axis_name="c", subcore_axis_name="s",
                                  num_cores=NC, num_subcores=NS)
    run = pl.kernel(
        _sc_body,
        out_type=jax.ShapeDtypeStruct((N, D), jnp.float32),
        mesh=mesh,
        compiler_params=pltpu.CompilerParams(needs_layout_passes=False),
        scratch_types=[
            pltpu.VMEM((TPW,), jnp.int32),
            pltpu.VMEM((TPW // 2,), jnp.int32),
            pltpu.VMEM((CB, DW), jnp.float32),
            pltpu.VMEM((CB, DW), jnp.float32),
            pltpu.VMEM((PB, DW), jnp.float32),
            pltpu.VMEM((PB, DW), jnp.float32),
            pltpu.VMEM((CB, D), jnp.float32),
            pltpu.VMEM((CB, D), jnp.float32),
            pltpu.VMEM((D,), jnp.float32),
            pltpu.VMEM((D,), jnp.float32),
            pltpu.SemaphoreType.DMA,
            pltpu.SemaphoreType.DMA,
            pltpu.SemaphoreType.DMA,
            pltpu.SemaphoreType.DMA,
        ],
    )
    out = run(words_f, pidx2, word_table, ptab2, gamma, beta)
    return out.reshape(B, L, D)


# speedup vs baseline: 3.6379x; 1.3371x over previous
import functools

import jax
import jax.numpy as jnp
from jax import lax
from jax.experimental import pallas as pl
from jax.experimental.pallas import tpu as pltpu
from jax.experimental.pallas import tpu_sc as plsc

DW, DP, D = 128, 64, 192
VOCAB_POS = 64
EPS = 1e-6
NC, NS = 2, 16
NW = NC * NS
B, L = 4096, 50
N = B * L
SPW = B // NW
CS = 4
CB = CS * L
PB = CB // 2
PBP = 104
PPS = L // 2
NCHUNK = SPW // CS
NJW = DW // 16
NJP = DP // 16
NJ = D // 16


def _norm_one(vals, gamma_v, beta_v, ob, row):
    s = vals[0]
    sq = vals[0] * vals[0]
    for v in vals[1:]:
        s = s + v
        sq = sq + v * v
    mean = jnp.sum(s) * (1.0 / D)
    var = jnp.sum(sq) * (1.0 / D) - mean * mean
    meanv = lax.broadcast(mean, (16,))
    xv = lax.broadcast(var + EPS, (16,))
    i = plsc.bitcast(xv, jnp.int32)
    i = 0x5F3759DF - lax.shift_right_arithmetic(i, 1)
    y = plsc.bitcast(i, jnp.float32)
    y = y * (1.5 - 0.5 * xv * y * y)
    y = y * (1.5 - 0.5 * xv * y * y)
    for j in range(NJ):
        g = gamma_v[pl.ds(16 * j, 16)]
        bt = beta_v[pl.ds(16 * j, 16)]
        ob[row, pl.ds(16 * j, 16)] = (vals[j] - meanv) * y * g + bt


def _sc_body(words_hbm, pidx2_hbm, wtab_hbm, ptab2_hbm, gamma_hbm, beta_hbm,
             out_hbm, widx_all, pidx_all, wrows0, wrows1, prows0, prows1,
             obuf0, obuf1, gamma_v, beta_v, sem0, sem1, osem0, osem1):
    wid = lax.axis_index("s") * NC + lax.axis_index("c")
    base = wid * SPW * L
    sbase0 = wid * SPW

    pltpu.sync_copy(gamma_hbm, gamma_v)
    pltpu.sync_copy(beta_hbm, beta_v)
    pltpu.sync_copy(words_hbm.at[pl.ds(base, SPW * L)], widx_all)
    pltpu.sync_copy(pidx2_hbm.at[wid], pidx_all)

    slot0 = (wrows0, prows0, sem0)
    slot1 = (wrows1, prows1, sem1)

    def _issue(c, slot):
        wrows, prows, sem = slot
        pltpu.async_copy(
            wtab_hbm.at[widx_all.at[pl.ds(c * CB, CB)]], wrows, sem)
        pltpu.async_copy(
            ptab2_hbm.at[pidx_all.at[c, pl.ds(0, PB)]], prows, sem)

    def _drain(c, slot):
        wrows, prows, sem = slot
        pltpu.make_async_copy(
            wtab_hbm.at[widx_all.at[pl.ds(c * CB, CB)]], wrows, sem).wait()
        pltpu.make_async_copy(
            ptab2_hbm.at[pidx_all.at[c, pl.ds(0, PB)]], prows, sem).wait()

    def _compute(c, slot, pending):
        wrows_v, prows_v, _ = slot
        sb = sbase0 + c * CS
        for si in range(CS):
            ob, osem = (obuf0, osem0) if si % 2 == 0 else (obuf1, osem1)
            if si >= 2 or pending is True:
                pltpu.make_async_copy(ob, out_hbm.at[sbase0], osem).wait()
            else:
                @pl.when(pending)
                def _():
                    pltpu.make_async_copy(ob, out_hbm.at[sbase0],
                                          osem).wait()

            @plsc.parallel_loop(0, PPS, unroll=5)
            def _pair(qq):
                q = si * PPS + qq
                x0 = [wrows_v[2 * q, pl.ds(16 * j, 16)] for j in range(NJW)]
                x1 = [wrows_v[2 * q + 1, pl.ds(16 * j, 16)]
                      for j in range(NJW)]
                p01 = [prows_v[q, pl.ds(16 * j, 16)] for j in range(2 * NJP)]
                _norm_one(x0 + p01[:NJP], gamma_v, beta_v, ob, 2 * qq)
                _norm_one(x1 + p01[NJP:], gamma_v, beta_v, ob, 2 * qq + 1)

            pltpu.async_copy(ob, out_hbm.at[sb + si], osem)

    _issue(0, slot0)

    @pl.loop(0, NCHUNK, step=2)
    def _chunk(c):
        _issue(c + 1, slot1)
        _drain(c, slot0)
        _compute(c, slot0, c > 0)

        @pl.when(c + 2 < NCHUNK)
        def _():
            _issue(c + 2, slot0)

        _drain(c + 1, slot1)
        _compute(c + 1, slot1, True)

    pltpu.make_async_copy(obuf0, out_hbm.at[sbase0], osem0).wait()
    pltpu.make_async_copy(obuf1, out_hbm.at[sbase0], osem1).wait()


TPW = SPW * L


@jax.jit
def kernel(words, postags, word_table, pos_table, gamma, beta):
    words_f = words.reshape(-1).astype(jnp.int32)
    pp = postags.reshape(-1, 2).astype(jnp.int32)
    pidx2 = pp[:, 0] * VOCAB_POS + pp[:, 1]
    pidx2 = jnp.pad(pidx2.reshape(NW, NCHUNK, PB),
                    ((0, 0), (0, 0), (0, PBP - PB)))
    ptab2 = jnp.concatenate(
        [jnp.broadcast_to(pos_table[:, None, :],
                          (VOCAB_POS, VOCAB_POS, DP)).reshape(-1, DP),
         jnp.broadcast_to(pos_table[None, :, :],
                          (VOCAB_POS, VOCAB_POS, DP)).reshape(-1, DP)],
        axis=1)
    mesh = plsc.VectorSubcoreMesh(core_axis_name="c", subcore_axis_name="s",
                                  num_cores=NC, num_subcores=NS)
    run = pl.kernel(
        _sc_body,
        out_type=jax.ShapeDtypeStruct((B, L, D), jnp.float32),
        mesh=mesh,
        compiler_params=pltpu.CompilerParams(needs_layout_passes=False),
        scratch_types=[
            pltpu.VMEM((TPW,), jnp.int32),
            pltpu.VMEM((NCHUNK, PBP), jnp.int32),
            pltpu.VMEM((CB, DW), jnp.float32),
            pltpu.VMEM((CB, DW), jnp.float32),
            pltpu.VMEM((PB, DW), jnp.float32),
            pltpu.VMEM((PB, DW), jnp.float32),
            pltpu.VMEM((L, D), jnp.float32),
            pltpu.VMEM((L, D), jnp.float32),
            pltpu.VMEM((D,), jnp.float32),
            pltpu.VMEM((D,), jnp.float32),
            pltpu.SemaphoreType.DMA,
            pltpu.SemaphoreType.DMA,
            pltpu.SemaphoreType.DMA,
            pltpu.SemaphoreType.DMA,
        ],
    )
    return run(words_f, pidx2, word_table, ptab2, gamma, beta)


# speedup vs baseline: 3.6433x; 1.0015x over previous
import functools

import jax
import jax.numpy as jnp
from jax import lax
from jax.experimental import pallas as pl
from jax.experimental.pallas import tpu as pltpu
from jax.experimental.pallas import tpu_sc as plsc

DW, DP, D = 128, 64, 192
VOCAB_POS = 64
EPS = 1e-6
NC, NS = 2, 16
NW = NC * NS
B, L = 4096, 50
N = B * L
SPW = B // NW
CS = 4
CB = CS * L
PB = CB // 2
PBP = 104
PPS = L // 2
NCHUNK = SPW // CS
NJW = DW // 16
NJP = DP // 16
NJ = D // 16


def _norm_one(vals, gamma_v, beta_v, ob, row):
    s = vals[0]
    sq = vals[0] * vals[0]
    for v in vals[1:]:
        s = s + v
        sq = sq + v * v
    mean = jnp.sum(s) * (1.0 / D)
    var = jnp.sum(sq) * (1.0 / D) - mean * mean
    meanv = lax.broadcast(mean, (16,))
    xv = lax.broadcast(var + EPS, (16,))
    i = plsc.bitcast(xv, jnp.int32)
    i = 0x5F3759DF - lax.shift_right_arithmetic(i, 1)
    y = plsc.bitcast(i, jnp.float32)
    y = y * (1.5 - 0.5 * xv * y * y)
    y = y * (1.5 - 0.5 * xv * y * y)
    for j in range(NJ):
        g = gamma_v[pl.ds(16 * j, 16)]
        bt = beta_v[pl.ds(16 * j, 16)]
        ob[row, pl.ds(16 * j, 16)] = (vals[j] - meanv) * y * g + bt


def _sc_body(words_hbm, pidx2_hbm, wtab_hbm, ptab2_hbm, gamma_hbm, beta_hbm,
             out_hbm, widx_all, pidx_all, wrows0, wrows1, prows0, prows1,
             obuf0, obuf1, gamma_v, beta_v, sem0, sem1, osem0, osem1):
    wid = lax.axis_index("s") * NC + lax.axis_index("c")
    base = wid * SPW * L
    sbase0 = wid * SPW

    pltpu.sync_copy(gamma_hbm, gamma_v)
    pltpu.sync_copy(beta_hbm, beta_v)
    pltpu.sync_copy(words_hbm.at[pl.ds(base, SPW * L)], widx_all)
    pltpu.sync_copy(pidx2_hbm.at[wid], pidx_all)

    slot0 = (wrows0, prows0, sem0)
    slot1 = (wrows1, prows1, sem1)

    def _issue(c, slot):
        wrows, prows, sem = slot
        pltpu.async_copy(
            wtab_hbm.at[widx_all.at[pl.ds(c * CB, CB)]], wrows, sem)
        pltpu.async_copy(
            ptab2_hbm.at[pidx_all.at[c, pl.ds(0, PB)]], prows, sem)

    def _drain(c, slot):
        wrows, prows, sem = slot
        pltpu.make_async_copy(
            wtab_hbm.at[widx_all.at[pl.ds(c * CB, CB)]], wrows, sem).wait()
        pltpu.make_async_copy(
            ptab2_hbm.at[pidx_all.at[c, pl.ds(0, PB)]], prows, sem).wait()

    def _compute(c, slot, pending):
        wrows_v, prows_v, _ = slot
        sb = sbase0 + c * CS
        for si in range(CS):
            ob, osem = (obuf0, osem0) if si % 2 == 0 else (obuf1, osem1)
            if si >= 2 or pending is True:
                pltpu.make_async_copy(ob, out_hbm.at[sbase0], osem).wait()
            else:
                @pl.when(pending)
                def _():
                    pltpu.make_async_copy(ob, out_hbm.at[sbase0],
                                          osem).wait()

            @plsc.parallel_loop(0, PPS, unroll=5)
            def _pair(qq):
                q = si * PPS + qq
                x0 = [wrows_v[2 * q, pl.ds(16 * j, 16)] for j in range(NJW)]
                x1 = [wrows_v[2 * q + 1, pl.ds(16 * j, 16)]
                      for j in range(NJW)]
                p01 = [prows_v[q, pl.ds(16 * j, 16)] for j in range(2 * NJP)]
                _norm_one(x0 + p01[:NJP], gamma_v, beta_v, ob, 2 * qq)
                _norm_one(x1 + p01[NJP:], gamma_v, beta_v, ob, 2 * qq + 1)

            pltpu.async_copy(ob, out_hbm.at[sb + si], osem)

    _issue(0, slot0)

    @pl.loop(0, NCHUNK, step=2)
    def _chunk(c):
        _issue(c + 1, slot1)
        _drain(c, slot0)
        _compute(c, slot0, c > 0)

        @pl.when(c + 2 < NCHUNK)
        def _():
            _issue(c + 2, slot0)

        _drain(c + 1, slot1)
        _compute(c + 1, slot1, True)

    pltpu.make_async_copy(obuf0, out_hbm.at[sbase0], osem0).wait()
    pltpu.make_async_copy(obuf1, out_hbm.at[sbase0], osem1).wait()


TPW = SPW * L


@jax.jit
def kernel(words, postags, word_table, pos_table, gamma, beta):
    words_f = words.reshape(-1).astype(jnp.int32)
    pp = postags.reshape(-1, 2).astype(jnp.int32)
    pidx2 = pp[:, 0] * VOCAB_POS + pp[:, 1]
    pidx2 = jnp.pad(pidx2.reshape(NW, NCHUNK, PB),
                    ((0, 0), (0, 0), (0, PBP - PB)))
    ptab2 = jnp.concatenate(
        [jnp.broadcast_to(pos_table[:, None, :],
                          (VOCAB_POS, VOCAB_POS, DP)).reshape(-1, DP),
         jnp.broadcast_to(pos_table[None, :, :],
                          (VOCAB_POS, VOCAB_POS, DP)).reshape(-1, DP)],
        axis=1)
    mesh = plsc.VectorSubcoreMesh(core_axis_name="c", subcore_axis_name="s",
                                  num_cores=NC, num_subcores=NS)
    run = pl.kernel(
        _sc_body,
        out_type=jax.ShapeDtypeStruct((B, L, D), jnp.float32),
        mesh=mesh,
        compiler_params=pltpu.CompilerParams(needs_layout_passes=False,
                                             use_tc_tiling_on_sc=True),
        scratch_types=[
            pltpu.VMEM((TPW,), jnp.int32),
            pltpu.VMEM((NCHUNK, PBP), jnp.int32),
            pltpu.VMEM((CB, DW), jnp.float32),
            pltpu.VMEM((CB, DW), jnp.float32),
            pltpu.VMEM((PB, DW), jnp.float32),
            pltpu.VMEM((PB, DW), jnp.float32),
            pltpu.VMEM((L, D), jnp.float32),
            pltpu.VMEM((L, D), jnp.float32),
            pltpu.VMEM((D,), jnp.float32),
            pltpu.VMEM((D,), jnp.float32),
            pltpu.SemaphoreType.DMA,
            pltpu.SemaphoreType.DMA,
            pltpu.SemaphoreType.DMA,
            pltpu.SemaphoreType.DMA,
        ],
    )
    return run(words_f, pidx2, word_table, ptab2, gamma, beta)
